# Initial kernel scaffold; baseline (speedup 1.0000x reference)
import jax
import jax.numpy as jnp
from jax.experimental import pallas as pl


def kernel(x, pos, batch, W, b):
    raise NotImplementedError("write your pallas kernel here")


# speedup vs baseline: 4.1972x; 4.1972x over previous
import functools

import jax
import jax.numpy as jnp
from jax.experimental import pallas as pl
from jax.experimental.pallas import tpu as pltpu

_B = 16


def _fused_mlp_segmax(x_ref, pos_ref, bat_ref, w1_ref, w2_ref, bias_ref,
                      out_ref, *, tile: int):
    i = pl.program_id(0)

    @pl.when(i == 0)
    def _init():
        out_ref[:] = jnp.full_like(out_ref, -jnp.inf)

    h = jnp.dot(x_ref[:], w1_ref[:], preferred_element_type=jnp.float32)
    h = h + jnp.dot(pos_ref[:], w2_ref[:], preferred_element_type=jnp.float32)
    h = jnp.maximum(h + bias_ref[:], 0.0)

    bt = bat_ref[:]
    first = bt[0, 0, 0]
    last = bt[0, 0, tile - 1]
    row = jax.lax.broadcasted_iota(jnp.int32, (tile, 1), 0)
    for s in range(_B):
        @pl.when(jnp.logical_and(first <= s, s <= last))
        def _seg(s=s):
            lo = jnp.sum((bt < s).astype(jnp.int32))
            hi = jnp.sum((bt <= s).astype(jnp.int32))
            m = jnp.logical_and(row >= lo, row < hi)
            seg = jnp.max(jnp.where(m, h, -jnp.inf), axis=0, keepdims=True)
            out_ref[s:s + 1, :] = jnp.maximum(out_ref[s:s + 1, :], seg)


def kernel(x, pos, batch, W, b):
    n, d = x.shape
    tile = 2000
    grid = n // tile

    w1 = W[:d]
    w2 = W[d:]
    bias = b.reshape(1, d)
    bat3 = batch.astype(jnp.int32).reshape(grid, 1, tile)

    pooled = pl.pallas_call(
        functools.partial(_fused_mlp_segmax, tile=tile),
        grid=(grid,),
        in_specs=[
            pl.BlockSpec((tile, d), lambda i: (i, 0)),
            pl.BlockSpec((tile, 3), lambda i: (i, 0)),
            pl.BlockSpec((1, 1, tile), lambda i: (i, 0, 0)),
            pl.BlockSpec((d, d), lambda i: (0, 0)),
            pl.BlockSpec((3, d), lambda i: (0, 0)),
            pl.BlockSpec((1, d), lambda i: (0, 0)),
        ],
        out_specs=pl.BlockSpec((_B, d), lambda i: (0, 0)),
        out_shape=jax.ShapeDtypeStruct((_B, d), jnp.float32),
    )(x, pos, bat3, w1, w2, bias)

    pos_out = jnp.zeros((_B, 3), dtype=pos.dtype)
    batch_out = jnp.arange(_B, dtype=jnp.int64)
    return (pooled, pos_out, batch_out)


# speedup vs baseline: 4.9589x; 1.1815x over previous
import functools

import jax
import jax.numpy as jnp
from jax.experimental import pallas as pl
from jax.experimental.pallas import tpu as pltpu

_B = 16


def _fused_mlp_segmax(x_ref, pos_ref, bat_ref, w1_ref, w2_ref, bias_ref,
                      out_ref, *, tile: int):
    i = pl.program_id(0)

    @pl.when(i == 0)
    def _init():
        out_ref[:] = jnp.full_like(out_ref, -jnp.inf)

    h = jnp.dot(x_ref[:], w1_ref[:], preferred_element_type=jnp.float32)
    h = h + jnp.dot(pos_ref[:], w2_ref[:], preferred_element_type=jnp.float32)
    h = jnp.maximum(h + bias_ref[:], 0.0)

    bt = bat_ref[:]
    first = bt[0, 0, 0]
    last = bt[0, 0, tile - 1]
    row = jax.lax.broadcasted_iota(jnp.int32, (tile, 1), 0)
    for s in range(_B):
        @pl.when(jnp.logical_and(first <= s, s <= last))
        def _seg(s=s):
            lo = jnp.sum((bt < s).astype(jnp.int32))
            hi = jnp.sum((bt <= s).astype(jnp.int32))
            m = jnp.logical_and(row >= lo, row < hi)
            seg = jnp.max(jnp.where(m, h, -jnp.inf), axis=0, keepdims=True)
            out_ref[s:s + 1, :] = jnp.maximum(out_ref[s:s + 1, :], seg)


def kernel(x, pos, batch, W, b):
    n, d = x.shape
    tile = 4000
    grid = n // tile

    w1 = W[:d]
    w2 = W[d:]
    bias = b.reshape(1, d)
    bat3 = batch.astype(jnp.int32).reshape(grid, 1, tile)

    pooled = pl.pallas_call(
        functools.partial(_fused_mlp_segmax, tile=tile),
        grid=(grid,),
        in_specs=[
            pl.BlockSpec((tile, d), lambda i: (i, 0)),
            pl.BlockSpec((tile, 3), lambda i: (i, 0)),
            pl.BlockSpec((1, 1, tile), lambda i: (i, 0, 0)),
            pl.BlockSpec((d, d), lambda i: (0, 0)),
            pl.BlockSpec((3, d), lambda i: (0, 0)),
            pl.BlockSpec((1, d), lambda i: (0, 0)),
        ],
        out_specs=pl.BlockSpec((_B, d), lambda i: (0, 0)),
        out_shape=jax.ShapeDtypeStruct((_B, d), jnp.float32),
    )(x, pos, bat3, w1, w2, bias)

    pos_out = jnp.zeros((_B, 3), dtype=pos.dtype)
    batch_out = jnp.arange(_B, dtype=jnp.int64)
    return (pooled, pos_out, batch_out)
